# baseline (speedup 1.0000x reference)
import functools

import jax
import jax.numpy as jnp
from jax import lax
from jax.experimental import pallas as pl
from jax.experimental.pallas import tpu as pltpu
from jax.experimental.pallas import tpu_sc as plsc

_INFO = plsc.get_sparse_core_info()
_NC = _INFO.num_cores
_NS = _INFO.num_subcores
_L = _INFO.num_lanes
_NW = _NC * _NS

_W = 128


def _make_kernel(n_e, s, d):
    assert n_e % (_W * _NW) == 0 or n_e == _W * _NW
    assert s % 2 == 0 and d % _L == 0

    mesh = plsc.VectorSubcoreMesh(core_axis_name="c", subcore_axis_name="s")

    @functools.partial(
        pl.kernel,
        mesh=mesh,
        compiler_params=pltpu.CompilerParams(
            use_tc_tiling_on_sc=False, needs_layout_passes=False
        ),
        out_type=jax.ShapeDtypeStruct((s * d, n_e), jnp.float32),
        scratch_types=[
            pltpu.VMEM((s, _W), jnp.int32),
            pltpu.VMEM((2, _W, d), jnp.float32),
            pltpu.VMEM((2, d, _W), jnp.float32),
            pltpu.SemaphoreType.DMA,
            pltpu.SemaphoreType.DMA,
            pltpu.SemaphoreType.DMA,
            pltpu.SemaphoreType.DMA,
        ],
    )
    def emb(idx_hbm, table_hbm, out_hbm, idx_wv, rows_v, slab_v,
            g0, g1, o0, o1):
        wid = lax.axis_index("s") * _NC + lax.axis_index("c")
        i0 = wid * _W
        gsem = (g0, g1)
        osem = (o0, o1)
        iota16 = lax.iota(jnp.int32, _L)

        pltpu.sync_copy(idx_hbm.at[:, pl.ds(i0, _W)], idx_wv)

        def fire_gather(j, b):
            pltpu.async_copy(
                table_hbm.at[idx_wv.at[j]], rows_v.at[b], gsem[b]
            )

        def drain_gather(j, b):
            pltpu.make_async_copy(
                table_hbm.at[idx_wv.at[j]], rows_v.at[b], gsem[b]
            ).wait()

        def fire_wb(j, b):
            pltpu.async_copy(
                slab_v.at[b],
                out_hbm.at[pl.ds(j * d, d), pl.ds(i0, _W)],
                osem[b],
            )

        def drain_wb(j, b):
            pltpu.make_async_copy(
                slab_v.at[b],
                out_hbm.at[pl.ds(j * d, d), pl.ds(i0, _W)],
                osem[b],
            ).wait()

        def transpose_mask(j, b):
            def outer(i16, carry):
                idxv = idx_wv[j, pl.ds(i16 * _L, _L)]
                fvec = jnp.minimum(idxv, 1).astype(jnp.float32)
                row_idx = i16 * _L + iota16
                for dd in range(d):
                    col = jnp.full((_L,), dd, jnp.int32)
                    vals = plsc.load_gather(rows_v.at[b], [row_idx, col])
                    slab_v[b, dd, pl.ds(i16 * _L, _L)] = vals * fvec
                return carry

            lax.fori_loop(0, _W // _L, outer, 0)

        fire_gather(0, 0)

        def body(jj, carry):
            for b in (0, 1):
                j = jj * 2 + b

                @pl.when(j + 1 < s)
                def _():
                    fire_gather(j + 1, 1 - b)

                drain_gather(j, b)

                @pl.when(j >= 2)
                def _():
                    drain_wb(j - 2, b)

                transpose_mask(j, b)
                fire_wb(j, b)
            return carry

        lax.fori_loop(0, s // 2, body, 0)
        drain_wb(s - 2, 0)
        drain_wb(s - 1, 1)

    return emb


def kernel(embedding_property, table):
    n_e, s = embedding_property.shape
    n_emb, d = table.shape
    idx_t = jnp.transpose(embedding_property).astype(jnp.int32)
    out2d = _make_kernel(n_e, s, d)(idx_t, table)
    return jnp.transpose(out2d.reshape(s, d, n_e), (2, 0, 1))


# speedup vs baseline: 1.5931x; 1.5931x over previous
import functools

import jax
import jax.numpy as jnp
from jax import lax
from jax.experimental import pallas as pl
from jax.experimental.pallas import tpu as pltpu
from jax.experimental.pallas import tpu_sc as plsc

_INFO = plsc.get_sparse_core_info()
_NC = _INFO.num_cores
_NS = _INFO.num_subcores
_L = _INFO.num_lanes
_NW = _NC * _NS

_W = 128


def _make_kernel(n_e, s, d):
    assert n_e % (_W * _NW) == 0 or n_e == _W * _NW
    assert s % 2 == 0 and d % _L == 0

    mesh = plsc.VectorSubcoreMesh(core_axis_name="c", subcore_axis_name="s")

    @functools.partial(
        pl.kernel,
        mesh=mesh,
        compiler_params=pltpu.CompilerParams(
            use_tc_tiling_on_sc=False, needs_layout_passes=False
        ),
        out_type=jax.ShapeDtypeStruct((s * d, n_e), jnp.float32),
        scratch_types=[
            pltpu.VMEM((s, _W), jnp.int32),
            pltpu.VMEM((2, _W, d), jnp.float32),
            pltpu.VMEM((2, d, _W), jnp.float32),
            pltpu.SemaphoreType.DMA,
            pltpu.SemaphoreType.DMA,
            pltpu.SemaphoreType.DMA,
            pltpu.SemaphoreType.DMA,
        ],
    )
    def emb(idx_hbm, table_hbm, out_hbm, idx_wv, rows_v, slab_v,
            g0, g1, o0, o1):
        wid = lax.axis_index("s") * _NC + lax.axis_index("c")
        i0 = wid * _W
        gsem = (g0, g1)
        osem = (o0, o1)
        iota16 = lax.iota(jnp.int32, _L)

        pltpu.sync_copy(idx_hbm.at[:, pl.ds(i0, _W)], idx_wv)

        def fire_gather(j, b):
            pltpu.async_copy(
                table_hbm.at[idx_wv.at[j]], rows_v.at[b], gsem[b]
            )

        def drain_gather(j, b):
            pltpu.make_async_copy(
                table_hbm.at[idx_wv.at[j]], rows_v.at[b], gsem[b]
            ).wait()

        def fire_wb(j, b):
            pltpu.async_copy(
                slab_v.at[b],
                out_hbm.at[pl.ds(j * d, d), pl.ds(i0, _W)],
                osem[b],
            )

        def drain_wb(j, b):
            pltpu.make_async_copy(
                slab_v.at[b],
                out_hbm.at[pl.ds(j * d, d), pl.ds(i0, _W)],
                osem[b],
            ).wait()

        row_idx = [i16 * _L + iota16 for i16 in range(_W // _L)]

        def transpose_mask(j, b):
            fvecs = [
                jnp.minimum(idx_wv[j, pl.ds(i16 * _L, _L)], 1).astype(
                    jnp.float32
                )
                for i16 in range(_W // _L)
            ]

            @plsc.parallel_loop(0, d, 1, unroll=4)
            def _(dd):
                col = jnp.full((_L,), dd, jnp.int32)
                for i16 in range(_W // _L):
                    vals = plsc.load_gather(
                        rows_v.at[b], [row_idx[i16], col]
                    )
                    slab_v[b, dd, pl.ds(i16 * _L, _L)] = vals * fvecs[i16]

        fire_gather(0, 0)

        def body(jj, carry):
            for b in (0, 1):
                j = jj * 2 + b

                @pl.when(j + 1 < s)
                def _():
                    fire_gather(j + 1, 1 - b)

                drain_gather(j, b)

                @pl.when(j >= 2)
                def _():
                    drain_wb(j - 2, b)

                transpose_mask(j, b)
                fire_wb(j, b)
            return carry

        lax.fori_loop(0, s // 2, body, 0)
        drain_wb(s - 2, 0)
        drain_wb(s - 1, 1)

    return emb


def kernel(embedding_property, table):
    n_e, s = embedding_property.shape
    n_emb, d = table.shape
    idx_t = jnp.transpose(embedding_property).astype(jnp.int32)
    out2d = _make_kernel(n_e, s, d)(idx_t, table)
    return jnp.transpose(out2d.reshape(s, d, n_e), (2, 0, 1))


# speedup vs baseline: 1.6182x; 1.0158x over previous
import functools

import jax
import jax.numpy as jnp
from jax import lax
from jax.experimental import pallas as pl
from jax.experimental.pallas import tpu as pltpu
from jax.experimental.pallas import tpu_sc as plsc

_INFO = plsc.get_sparse_core_info()
_NC = _INFO.num_cores
_NS = _INFO.num_subcores
_L = _INFO.num_lanes
_NW = _NC * _NS

_W = 128


def _make_kernel(n_e, s, d):
    assert n_e % (_W * _NW) == 0 or n_e == _W * _NW
    assert s % 2 == 0 and d % _L == 0

    mesh = plsc.VectorSubcoreMesh(core_axis_name="c", subcore_axis_name="s")

    @functools.partial(
        pl.kernel,
        mesh=mesh,
        compiler_params=pltpu.CompilerParams(
            use_tc_tiling_on_sc=False, needs_layout_passes=False
        ),
        out_type=jax.ShapeDtypeStruct((s * d, n_e), jnp.float32),
        scratch_types=[
            pltpu.VMEM((s, _W), jnp.int32),
            pltpu.VMEM((2, _W, d), jnp.float32),
            pltpu.VMEM((2, d, _W + 4), jnp.float32),
            pltpu.SemaphoreType.DMA,
            pltpu.SemaphoreType.DMA,
            pltpu.SemaphoreType.DMA,
            pltpu.SemaphoreType.DMA,
        ],
    )
    def emb(idx_hbm, table_hbm, out_hbm, idx_wv, rows_v, slab_v,
            g0, g1, o0, o1):
        wid = lax.axis_index("s") * _NC + lax.axis_index("c")
        i0 = wid * _W
        gsem = (g0, g1)
        osem = (o0, o1)
        iota16 = lax.iota(jnp.int32, _L)

        pltpu.sync_copy(idx_hbm.at[:, pl.ds(i0, _W)], idx_wv)

        def fire_gather(j, b):
            pltpu.async_copy(
                table_hbm.at[idx_wv.at[j]], rows_v.at[b], gsem[b]
            )

        def drain_gather(j, b):
            pltpu.make_async_copy(
                table_hbm.at[idx_wv.at[j]], rows_v.at[b], gsem[b]
            ).wait()

        def fire_wb(j, b):
            pltpu.async_copy(
                slab_v.at[b, :, pl.ds(0, _W)],
                out_hbm.at[pl.ds(j * d, d), pl.ds(i0, _W)],
                osem[b],
            )

        def drain_wb(j, b):
            pltpu.make_async_copy(
                slab_v.at[b, :, pl.ds(0, _W)],
                out_hbm.at[pl.ds(j * d, d), pl.ds(i0, _W)],
                osem[b],
            ).wait()

        dd_rows = [dd16 * _L + iota16 for dd16 in range(d // _L)]

        def transpose_mask(j, b):
            @plsc.parallel_loop(0, _W // _L, 1, unroll=2)
            def _(i16):
                mvec = idx_wv[j, pl.ds(i16 * _L, _L)]
                fvec = jnp.minimum(mvec, 1).astype(jnp.float32)
                for r in range(_L):
                    fac = jnp.full((_L,), fvec[r], jnp.float32)
                    i = i16 * _L + r
                    col = jnp.full((_L,), i, jnp.int32)
                    for dd16 in range(d // _L):
                        v = rows_v[b, i, pl.ds(dd16 * _L, _L)]
                        plsc.store_scatter(
                            slab_v.at[b], [dd_rows[dd16], col], v * fac
                        )

        fire_gather(0, 0)

        def body(jj, carry):
            for b in (0, 1):
                j = jj * 2 + b

                @pl.when(j + 1 < s)
                def _():
                    fire_gather(j + 1, 1 - b)

                drain_gather(j, b)

                @pl.when(j >= 2)
                def _():
                    drain_wb(j - 2, b)

                transpose_mask(j, b)
                fire_wb(j, b)
            return carry

        lax.fori_loop(0, s // 2, body, 0)
        drain_wb(s - 2, 0)
        drain_wb(s - 1, 1)

    return emb


def kernel(embedding_property, table):
    n_e, s = embedding_property.shape
    n_emb, d = table.shape
    idx_t = jnp.transpose(embedding_property).astype(jnp.int32)
    out2d = _make_kernel(n_e, s, d)(idx_t, table)
    return jnp.transpose(out2d.reshape(s, d, n_e), (2, 0, 1))
